# Initial kernel scaffold; baseline (speedup 1.0000x reference)
import jax
import jax.numpy as jnp
from jax.experimental import pallas as pl


def kernel(x, edge_index, edge_index_cross, W1, b1, W2, b2, Wc1, bc1, Wc2, bc2):
    raise NotImplementedError("write your pallas kernel here")


# speedup vs baseline: 6.3740x; 6.3740x over previous
import functools

import jax
import jax.numpy as jnp
from jax import lax
from jax.experimental import pallas as pl
from jax.experimental.pallas import tpu as pltpu
from jax.experimental.pallas import tpu_sc as plsc

N = 10000
D = 128
NP = 10240
E = 320000
EP = 327680
K = 128
NW = 32
EPW = EP // NW
NBLK = EPW // K
RPT = NP // 16
PAD = N

_MESH = plsc.VectorSubcoreMesh(core_axis_name="c", subcore_axis_name="s")


def _zero_vec(ref, nelem):
  def body(i, _):
    ref[pl.ds(i * 16, 16)] = jnp.zeros((16,), jnp.float32)
    return _
  lax.fori_loop(0, nelem // 16, body, None)


def _deg_body(dst_hbm, out_hbm, dst_v, ones_v, zrow_v, dacc):
  c = lax.axis_index("c")
  s = lax.axis_index("s")
  w = c * 16 + s
  for j in range(K // 16):
    ones_v[pl.ds(j * 16, 16)] = jnp.ones((16,), jnp.float32)
  _zero_vec(zrow_v, RPT)
  pltpu.sync_copy(zrow_v, dacc.at[pl.ds(s * RPT, RPT)])
  plsc.subcore_barrier()
  base = w * EPW
  def blk(i, _):
    pltpu.sync_copy(dst_hbm.at[pl.ds(base + i * K, K)], dst_v)
    pltpu.sync_copy(ones_v, dacc.at[dst_v], add=True)
    return _
  lax.fori_loop(0, NBLK, blk, None)
  plsc.subcore_barrier()
  pltpu.sync_copy(dacc.at[pl.ds(s * RPT, RPT)], out_hbm.at[w])


_deg = pl.kernel(
    _deg_body,
    out_type=jax.ShapeDtypeStruct((NW, RPT), jnp.float32),
    mesh=_MESH,
    scratch_types=[
        pltpu.VMEM((K,), jnp.int32),
        pltpu.VMEM((K,), jnp.float32),
        pltpu.VMEM((RPT,), jnp.float32),
        pltpu.VMEM_SHARED((NP,), jnp.float32),
    ],
)


def _agg_body(hs_hbm, src_hbm, dst_hbm, out_hbm,
              src_v, dst_v, rows_v, zbuf, acc, sem):
  c = lax.axis_index("c")
  s = lax.axis_index("s")
  w = c * 16 + s
  def zfill(i, _):
    for j in range(D // 16):
      zbuf[i, pl.ds(j * 16, 16)] = jnp.zeros((16,), jnp.float32)
    return _
  lax.fori_loop(0, 32, zfill, None)
  def zcp(i, _):
    pltpu.sync_copy(zbuf, acc.at[pl.ds(s * RPT + i * 32, 32)])
    return _
  lax.fori_loop(0, RPT // 32, zcp, None)
  plsc.subcore_barrier()
  base = w * EPW
  def blk(i, _):
    off = base + i * K
    pltpu.sync_copy(src_hbm.at[pl.ds(off, K)], src_v)
    pltpu.sync_copy(dst_hbm.at[pl.ds(off, K)], dst_v)
    pltpu.async_copy(hs_hbm.at[src_v], rows_v, sem).wait()
    pltpu.sync_copy(rows_v, acc.at[dst_v], add=True)
    return _
  lax.fori_loop(0, NBLK, blk, None)
  plsc.subcore_barrier()
  pltpu.sync_copy(acc.at[pl.ds(s * RPT, RPT)], out_hbm.at[w])


_agg = pl.kernel(
    _agg_body,
    out_type=jax.ShapeDtypeStruct((NW, RPT, D), jnp.float32),
    mesh=_MESH,
    scratch_types=[
        pltpu.VMEM((K,), jnp.int32),
        pltpu.VMEM((K,), jnp.int32),
        pltpu.VMEM((K, D), jnp.float32),
        pltpu.VMEM((32, D), jnp.float32),
        pltpu.VMEM_SHARED((NP, D), jnp.float32),
        pltpu.SemaphoreType.DMA,
    ],
)


BM = 2048


def _mm2_body(x_ref, w_ref, d1_ref, d2_ref, o1_ref, o2_ref):
  h = jnp.dot(x_ref[...], w_ref[...], preferred_element_type=jnp.float32)
  o1_ref[...] = d1_ref[...] * h[:, :D]
  o2_ref[...] = d2_ref[...] * h[:, D:]


_mm2 = pl.pallas_call(
    _mm2_body,
    grid=(NP // BM,),
    in_specs=[
        pl.BlockSpec((BM, D), lambda i: (i, 0)),
        pl.BlockSpec((D, 2 * D), lambda i: (0, 0)),
        pl.BlockSpec((BM, 1), lambda i: (i, 0)),
        pl.BlockSpec((BM, 1), lambda i: (i, 0)),
    ],
    out_specs=[pl.BlockSpec((BM, D), lambda i: (i, 0))] * 2,
    out_shape=[jax.ShapeDtypeStruct((NP, D), jnp.float32)] * 2,
)


def _mid_body(p1_ref, hs1_ref, b1_ref, d1_ref, w2_ref,
              p1c_ref, hs1c_ref, bc1_ref, d2_ref, wc2_ref,
              o1_ref, o2_ref):
  t1 = jnp.maximum(
      d1_ref[...] * (p1_ref[0] + p1_ref[1] + hs1_ref[...]) + b1_ref[...], 0.0)
  o1_ref[...] = d1_ref[...] * jnp.dot(
      t1, w2_ref[...], preferred_element_type=jnp.float32)
  t2 = jnp.maximum(
      d2_ref[...] * (p1c_ref[0] + p1c_ref[1] + hs1c_ref[...]) + bc1_ref[...],
      0.0)
  o2_ref[...] = d2_ref[...] * jnp.dot(
      t2, wc2_ref[...], preferred_element_type=jnp.float32)


_mid = pl.pallas_call(
    _mid_body,
    grid=(NP // BM,),
    in_specs=[
        pl.BlockSpec((2, BM, D), lambda i: (0, i, 0)),
        pl.BlockSpec((BM, D), lambda i: (i, 0)),
        pl.BlockSpec((1, D), lambda i: (0, 0)),
        pl.BlockSpec((BM, 1), lambda i: (i, 0)),
        pl.BlockSpec((D, D), lambda i: (0, 0)),
        pl.BlockSpec((2, BM, D), lambda i: (0, i, 0)),
        pl.BlockSpec((BM, D), lambda i: (i, 0)),
        pl.BlockSpec((1, D), lambda i: (0, 0)),
        pl.BlockSpec((BM, 1), lambda i: (i, 0)),
        pl.BlockSpec((D, D), lambda i: (0, 0)),
    ],
    out_specs=[pl.BlockSpec((BM, D), lambda i: (i, 0))] * 2,
    out_shape=[jax.ShapeDtypeStruct((NP, D), jnp.float32)] * 2,
)


BF = 2000


def _fin_body(p2_ref, hs2_ref, b2_ref, d1_ref,
              p2c_ref, hs2c_ref, bc2_ref, d2_ref, o_ref):
  t1 = jnp.maximum(
      d1_ref[...] * (p2_ref[0] + p2_ref[1] + hs2_ref[...]) + b2_ref[...], 0.0)
  t2 = jnp.maximum(
      d2_ref[...] * (p2c_ref[0] + p2c_ref[1] + hs2c_ref[...]) + bc2_ref[...],
      0.0)
  o_ref[...] = jnp.concatenate([t1, t2], axis=1)


_fin = pl.pallas_call(
    _fin_body,
    grid=(N // BF,),
    in_specs=[
        pl.BlockSpec((2, BF, D), lambda i: (0, i, 0)),
        pl.BlockSpec((BF, D), lambda i: (i, 0)),
        pl.BlockSpec((1, D), lambda i: (0, 0)),
        pl.BlockSpec((BF, 1), lambda i: (i, 0)),
        pl.BlockSpec((2, BF, D), lambda i: (0, i, 0)),
        pl.BlockSpec((BF, D), lambda i: (i, 0)),
        pl.BlockSpec((1, D), lambda i: (0, 0)),
        pl.BlockSpec((BF, 1), lambda i: (i, 0)),
    ],
    out_specs=pl.BlockSpec((BF, 2 * D), lambda i: (i, 0)),
    out_shape=jax.ShapeDtypeStruct((N, 2 * D), jnp.float32),
)


def _prep_edges(ei):
  src = ei[0].astype(jnp.int32)
  dst = ei[1].astype(jnp.int32)
  pad = jnp.full((EP - E,), PAD, jnp.int32)
  return jnp.concatenate([src, pad]), jnp.concatenate([dst, pad])


def _mk_dinv(degp):
  deg = degp.reshape(2, NP).sum(0) + 1.0
  dinv = lax.rsqrt(deg)
  mask = (jnp.arange(NP) < N).astype(jnp.float32)
  return (dinv * mask)[:, None]


def kernel(x, edge_index, edge_index_cross, W1, b1, W2, b2,
           Wc1, bc1, Wc2, bc2):
  f32 = jnp.float32
  xp = jnp.zeros((NP, D), f32).at[:N].set(x.astype(f32))
  s1, t1 = _prep_edges(edge_index)
  s2, t2 = _prep_edges(edge_index_cross)

  dv1 = _mk_dinv(_deg(t1))
  dv2 = _mk_dinv(_deg(t2))

  Wcat = jnp.concatenate([W1.astype(f32), Wc1.astype(f32)], axis=1)
  hs1, hs1c = _mm2(xp, Wcat, dv1, dv2)

  p1 = _agg(hs1, s1, t1).reshape(2, NP, D)
  p1c = _agg(hs1c, s2, t2).reshape(2, NP, D)

  hs2, hs2c = _mid(p1, hs1, b1.reshape(1, D), dv1, W2.astype(f32),
                   p1c, hs1c, bc1.reshape(1, D), dv2, Wc2.astype(f32))

  p2 = _agg(hs2, s1, t1).reshape(2, NP, D)
  p2c = _agg(hs2c, s2, t2).reshape(2, NP, D)

  return _fin(p2, hs2, b2.reshape(1, D), dv1,
              p2c, hs2c, bc2.reshape(1, D), dv2)


# speedup vs baseline: 28.6669x; 4.4975x over previous
import functools

import jax
import jax.numpy as jnp
from jax import lax
from jax.experimental import pallas as pl
from jax.experimental.pallas import tpu as pltpu
from jax.experimental.pallas import tpu_sc as plsc

N = 10000
D = 128
NP = 10240
E = 320000
EP = 327680
K = 128
NW = 32
EPW = EP // NW
NBLK = EPW // K
RPT = NP // 16
PAD = N

_MESH = plsc.VectorSubcoreMesh(core_axis_name="c", subcore_axis_name="s")


def _zero_vec(ref, nelem):
  def body(i, _):
    ref[pl.ds(i * 16, 16)] = jnp.zeros((16,), jnp.float32)
    return _
  lax.fori_loop(0, nelem // 16, body, None)


def _deg_body(dst_hbm, out_hbm, dst_v, ones_v, zrow_v, dacc):
  c = lax.axis_index("c")
  s = lax.axis_index("s")
  w = c * 16 + s
  for j in range(K // 16):
    ones_v[pl.ds(j * 16, 16)] = jnp.ones((16,), jnp.float32)
  _zero_vec(zrow_v, RPT)
  pltpu.sync_copy(zrow_v, dacc.at[pl.ds(s * RPT, RPT)])
  pltpu.sync_copy(dst_hbm.at[pl.ds(w * NBLK, NBLK)], dst_v)
  plsc.subcore_barrier()
  def blk(i, _):
    pltpu.sync_copy(ones_v, dacc.at[dst_v.at[i]], add=True)
    return _
  lax.fori_loop(0, NBLK, blk, None)
  plsc.subcore_barrier()
  pltpu.sync_copy(dacc.at[pl.ds(s * RPT, RPT)], out_hbm.at[w])


_deg = pl.kernel(
    _deg_body,
    out_type=jax.ShapeDtypeStruct((NW, RPT), jnp.float32),
    mesh=_MESH,
    scratch_types=[
        pltpu.VMEM((NBLK, K), jnp.int32),
        pltpu.VMEM((K,), jnp.float32),
        pltpu.VMEM((RPT,), jnp.float32),
        pltpu.VMEM_SHARED((NP,), jnp.float32),
    ],
)


CHK = 16
NCHK = NBLK // CHK


def _agg_body(hs_hbm, src_hbm, dst_hbm, out_hbm,
              src_sl, dst_sl, rows_a, rows_b, acc,
              sem_a, sem_b, sem_ss, sem_sd):
  c = lax.axis_index("c")
  s = lax.axis_index("s")
  w = c * 16 + s
  base = w * NBLK
  def zfill(i, _):
    for j in range(D // 16):
      rows_a[i, pl.ds(j * 16, 16)] = jnp.zeros((16,), jnp.float32)
    return _
  lax.fori_loop(0, K, zfill, None)
  def zcp(i, _):
    pltpu.sync_copy(rows_a, acc.at[pl.ds(s * RPT + i * K, K)])
    return _
  lax.fori_loop(0, RPT // K, zcp, None)
  plsc.subcore_barrier()

  def slab_issue(ck, p):
    pltpu.async_copy(src_hbm.at[pl.ds(base + ck * CHK, CHK)],
                     src_sl.at[p], sem_ss)
    pltpu.async_copy(dst_hbm.at[pl.ds(base + ck * CHK, CHK)],
                     dst_sl.at[p], sem_sd)

  def slab_wait(ck, p):
    pltpu.make_async_copy(src_hbm.at[pl.ds(base + ck * CHK, CHK)],
                          src_sl.at[p], sem_ss).wait()
    pltpu.make_async_copy(dst_hbm.at[pl.ds(base + ck * CHK, CHK)],
                          dst_sl.at[p], sem_sd).wait()

  slab_issue(0, 0)
  for ck in range(NCHK):
    p = ck % 2
    slab_wait(ck, p)
    if ck + 1 < NCHK:
      slab_issue(ck + 1, (ck + 1) % 2)
    pltpu.async_copy(hs_hbm.at[src_sl.at[p, 0]], rows_a, sem_a)
    def pair(j, _):
      e = 2 * j
      pltpu.async_copy(hs_hbm.at[src_sl.at[p, e + 1]], rows_b, sem_b)
      pltpu.make_async_copy(hs_hbm.at[src_sl.at[p, e]], rows_a, sem_a).wait()
      pltpu.sync_copy(rows_a, acc.at[dst_sl.at[p, e]], add=True)
      nxt = jnp.minimum(e + 2, CHK - 1)
      pltpu.async_copy(hs_hbm.at[src_sl.at[p, nxt]], rows_a, sem_a)
      pltpu.make_async_copy(hs_hbm.at[src_sl.at[p, e + 1]], rows_b,
                            sem_b).wait()
      pltpu.sync_copy(rows_b, acc.at[dst_sl.at[p, e + 1]], add=True)
      return _
    lax.fori_loop(0, CHK // 2, pair, None)
    pltpu.make_async_copy(hs_hbm.at[src_sl.at[p, CHK - 1]], rows_a,
                          sem_a).wait()
  plsc.subcore_barrier()
  pltpu.sync_copy(acc.at[pl.ds(s * RPT, RPT)], out_hbm.at[w])


_agg = pl.kernel(
    _agg_body,
    out_type=jax.ShapeDtypeStruct((NW, RPT, D), jnp.float32),
    mesh=_MESH,
    scratch_types=[
        pltpu.VMEM((2, CHK, K), jnp.int32),
        pltpu.VMEM((2, CHK, K), jnp.int32),
        pltpu.VMEM((K, D), jnp.float32),
        pltpu.VMEM((K, D), jnp.float32),
        pltpu.VMEM_SHARED((NP, D), jnp.float32),
        pltpu.SemaphoreType.DMA,
        pltpu.SemaphoreType.DMA,
        pltpu.SemaphoreType.DMA,
        pltpu.SemaphoreType.DMA,
    ],
)


BM = 2048


def _mm2_body(x_ref, w_ref, d1_ref, d2_ref, o1_ref, o2_ref):
  h = jnp.dot(x_ref[...], w_ref[...], preferred_element_type=jnp.float32)
  o1_ref[...] = d1_ref[...] * h[:, :D]
  o2_ref[...] = d2_ref[...] * h[:, D:]


_mm2 = pl.pallas_call(
    _mm2_body,
    grid=(NP // BM,),
    in_specs=[
        pl.BlockSpec((BM, D), lambda i: (i, 0)),
        pl.BlockSpec((D, 2 * D), lambda i: (0, 0)),
        pl.BlockSpec((BM, 1), lambda i: (i, 0)),
        pl.BlockSpec((BM, 1), lambda i: (i, 0)),
    ],
    out_specs=[pl.BlockSpec((BM, D), lambda i: (i, 0))] * 2,
    out_shape=[jax.ShapeDtypeStruct((NP, D), jnp.float32)] * 2,
)


def _mid_body(p1_ref, hs1_ref, b1_ref, d1_ref, w2_ref,
              p1c_ref, hs1c_ref, bc1_ref, d2_ref, wc2_ref,
              o1_ref, o2_ref):
  t1 = jnp.maximum(
      d1_ref[...] * (p1_ref[0] + p1_ref[1] + hs1_ref[...]) + b1_ref[...], 0.0)
  o1_ref[...] = d1_ref[...] * jnp.dot(
      t1, w2_ref[...], preferred_element_type=jnp.float32)
  t2 = jnp.maximum(
      d2_ref[...] * (p1c_ref[0] + p1c_ref[1] + hs1c_ref[...]) + bc1_ref[...],
      0.0)
  o2_ref[...] = d2_ref[...] * jnp.dot(
      t2, wc2_ref[...], preferred_element_type=jnp.float32)


_mid = pl.pallas_call(
    _mid_body,
    grid=(NP // BM,),
    in_specs=[
        pl.BlockSpec((2, BM, D), lambda i: (0, i, 0)),
        pl.BlockSpec((BM, D), lambda i: (i, 0)),
        pl.BlockSpec((1, D), lambda i: (0, 0)),
        pl.BlockSpec((BM, 1), lambda i: (i, 0)),
        pl.BlockSpec((D, D), lambda i: (0, 0)),
        pl.BlockSpec((2, BM, D), lambda i: (0, i, 0)),
        pl.BlockSpec((BM, D), lambda i: (i, 0)),
        pl.BlockSpec((1, D), lambda i: (0, 0)),
        pl.BlockSpec((BM, 1), lambda i: (i, 0)),
        pl.BlockSpec((D, D), lambda i: (0, 0)),
    ],
    out_specs=[pl.BlockSpec((BM, D), lambda i: (i, 0))] * 2,
    out_shape=[jax.ShapeDtypeStruct((NP, D), jnp.float32)] * 2,
)


BF = 2000


def _fin_body(p2_ref, hs2_ref, b2_ref, d1_ref,
              p2c_ref, hs2c_ref, bc2_ref, d2_ref, o_ref):
  t1 = jnp.maximum(
      d1_ref[...] * (p2_ref[0] + p2_ref[1] + hs2_ref[...]) + b2_ref[...], 0.0)
  t2 = jnp.maximum(
      d2_ref[...] * (p2c_ref[0] + p2c_ref[1] + hs2c_ref[...]) + bc2_ref[...],
      0.0)
  o_ref[...] = jnp.concatenate([t1, t2], axis=1)


_fin = pl.pallas_call(
    _fin_body,
    grid=(N // BF,),
    in_specs=[
        pl.BlockSpec((2, BF, D), lambda i: (0, i, 0)),
        pl.BlockSpec((BF, D), lambda i: (i, 0)),
        pl.BlockSpec((1, D), lambda i: (0, 0)),
        pl.BlockSpec((BF, 1), lambda i: (i, 0)),
        pl.BlockSpec((2, BF, D), lambda i: (0, i, 0)),
        pl.BlockSpec((BF, D), lambda i: (i, 0)),
        pl.BlockSpec((1, D), lambda i: (0, 0)),
        pl.BlockSpec((BF, 1), lambda i: (i, 0)),
    ],
    out_specs=pl.BlockSpec((BF, 2 * D), lambda i: (i, 0)),
    out_shape=jax.ShapeDtypeStruct((N, 2 * D), jnp.float32),
)


def _prep_edges(ei):
  src = ei[0].astype(jnp.int32)
  dst = ei[1].astype(jnp.int32)
  pad = N + jnp.arange(EP - E, dtype=jnp.int32) % (NP - N)
  src2 = jnp.concatenate([src, pad]).reshape(EP // K, K)
  dst2 = jnp.concatenate([dst, pad]).reshape(EP // K, K)
  return src2, dst2


def _mk_dinv(degp):
  deg = degp.reshape(2, NP).sum(0) + 1.0
  dinv = lax.rsqrt(deg)
  mask = (jnp.arange(NP) < N).astype(jnp.float32)
  return (dinv * mask)[:, None]


def kernel(x, edge_index, edge_index_cross, W1, b1, W2, b2,
           Wc1, bc1, Wc2, bc2):
  f32 = jnp.float32
  xp = jnp.zeros((NP, D), f32).at[:N].set(x.astype(f32))
  s1, t1 = _prep_edges(edge_index)
  s2, t2 = _prep_edges(edge_index_cross)

  dv1 = _mk_dinv(_deg(t1))
  dv2 = _mk_dinv(_deg(t2))

  Wcat = jnp.concatenate([W1.astype(f32), Wc1.astype(f32)], axis=1)
  hs1, hs1c = _mm2(xp, Wcat, dv1, dv2)

  p1 = _agg(hs1, s1, t1).reshape(2, NP, D)
  p1c = _agg(hs1c, s2, t2).reshape(2, NP, D)

  hs2, hs2c = _mid(p1, hs1, b1.reshape(1, D), dv1, W2.astype(f32),
                   p1c, hs1c, bc1.reshape(1, D), dv2, Wc2.astype(f32))

  p2 = _agg(hs2, s1, t1).reshape(2, NP, D)
  p2c = _agg(hs2c, s2, t2).reshape(2, NP, D)

  return _fin(p2, hs2, b2.reshape(1, D), dv1,
              p2c, hs2c, bc2.reshape(1, D), dv2)


# speedup vs baseline: 29.9365x; 1.0443x over previous
import functools

import jax
import jax.numpy as jnp
from jax import lax
from jax.experimental import pallas as pl
from jax.experimental.pallas import tpu as pltpu
from jax.experimental.pallas import tpu_sc as plsc

N = 10000
D = 128
NP = 10240
E = 320000
EP = 327680
K = 128
NBLKS = EP // K
TBLK = NBLKS // 16
CHK = 16
NCHK = TBLK // CHK
RPT = NP // 16

_MESH = plsc.VectorSubcoreMesh(core_axis_name="c", subcore_axis_name="s")


def _deg_body(dst_hbm, out_hbm, dst_sl, ones_v, zrow_v, dacc):
  c = lax.axis_index("c")
  s = lax.axis_index("s")
  w = c * 16 + s
  for j in range(K // 16):
    ones_v[pl.ds(j * 16, 16)] = jnp.ones((16,), jnp.float32)
  def zfill(i, _):
    zrow_v[pl.ds(i * 16, 16)] = jnp.zeros((16,), jnp.float32)
    return _
  lax.fori_loop(0, RPT // 16, zfill, None)
  pltpu.sync_copy(zrow_v, dacc.at[pl.ds(s * RPT, RPT)])
  pltpu.sync_copy(dst_hbm.at[pl.ds(w * TBLK, TBLK)], dst_sl)
  plsc.subcore_barrier()
  def blk(i, _):
    pltpu.sync_copy(ones_v, dacc.at[dst_sl.at[i]], add=True)
    return _
  lax.fori_loop(0, TBLK, blk, None)
  plsc.subcore_barrier()
  pltpu.sync_copy(dacc.at[pl.ds(s * RPT, RPT)], out_hbm.at[w])


_deg = pl.kernel(
    _deg_body,
    out_type=jax.ShapeDtypeStruct((32, RPT), jnp.float32),
    mesh=_MESH,
    scratch_types=[
        pltpu.VMEM((TBLK, K), jnp.int32),
        pltpu.VMEM((K,), jnp.float32),
        pltpu.VMEM((RPT,), jnp.float32),
        pltpu.VMEM_SHARED((NP,), jnp.float32),
    ],
)


def _agg_body(hs_hbm, src_hbm, dst_hbm, out_hbm,
              src_sl, dst_sl, rows_a, rows_b, acc,
              sem_a, sem_b, sem_ss, sem_sd):
  c = lax.axis_index("c")
  s = lax.axis_index("s")
  w = c * 16 + s
  base = w * TBLK
  def zfill(i, _):
    for j in range(D // 16):
      rows_a[i, pl.ds(j * 16, 16)] = jnp.zeros((16,), jnp.float32)
    return _
  lax.fori_loop(0, K, zfill, None)
  def zcp(i, _):
    pltpu.sync_copy(rows_a, acc.at[pl.ds(s * RPT + i * K, K)])
    return _
  lax.fori_loop(0, RPT // K, zcp, None)
  plsc.subcore_barrier()

  def slab_issue(ck, p):
    pltpu.async_copy(src_hbm.at[pl.ds(base + ck * CHK, CHK)],
                     src_sl.at[p], sem_ss)
    pltpu.async_copy(dst_hbm.at[pl.ds(base + ck * CHK, CHK)],
                     dst_sl.at[p], sem_sd)

  def slab_wait(ck, p):
    pltpu.make_async_copy(src_hbm.at[pl.ds(base + ck * CHK, CHK)],
                          src_sl.at[p], sem_ss).wait()
    pltpu.make_async_copy(dst_hbm.at[pl.ds(base + ck * CHK, CHK)],
                          dst_sl.at[p], sem_sd).wait()

  slab_issue(0, 0)
  def chunk(ck, _):
    p = ck % 2
    slab_wait(ck, p)
    @pl.when(ck + 1 < NCHK)
    def _issue_next():
      slab_issue(ck + 1, (ck + 1) % 2)
    pltpu.async_copy(hs_hbm.at[src_sl.at[p, 0]], rows_a, sem_a)
    def pair(j, _2):
      e = 2 * j
      pltpu.async_copy(hs_hbm.at[src_sl.at[p, e + 1]], rows_b, sem_b)
      pltpu.make_async_copy(hs_hbm.at[src_sl.at[p, e]], rows_a, sem_a).wait()
      pltpu.sync_copy(rows_a, acc.at[dst_sl.at[p, e]], add=True)
      nxt = jnp.minimum(e + 2, CHK - 1)
      pltpu.async_copy(hs_hbm.at[src_sl.at[p, nxt]], rows_a, sem_a)
      pltpu.make_async_copy(hs_hbm.at[src_sl.at[p, e + 1]], rows_b,
                            sem_b).wait()
      pltpu.sync_copy(rows_b, acc.at[dst_sl.at[p, e + 1]], add=True)
      return _2
    lax.fori_loop(0, CHK // 2, pair, None)
    pltpu.make_async_copy(hs_hbm.at[src_sl.at[p, CHK - 1]], rows_a,
                          sem_a).wait()
    return _
  lax.fori_loop(0, NCHK, chunk, None)
  plsc.subcore_barrier()
  pltpu.sync_copy(acc.at[pl.ds(s * RPT, RPT)], out_hbm.at[w])


_agg = pl.kernel(
    _agg_body,
    out_type=jax.ShapeDtypeStruct((32, RPT, D), jnp.float32),
    mesh=_MESH,
    scratch_types=[
        pltpu.VMEM((2, CHK, K), jnp.int32),
        pltpu.VMEM((2, CHK, K), jnp.int32),
        pltpu.VMEM((K, D), jnp.float32),
        pltpu.VMEM((K, D), jnp.float32),
        pltpu.VMEM_SHARED((NP, D), jnp.float32),
        pltpu.SemaphoreType.DMA,
        pltpu.SemaphoreType.DMA,
        pltpu.SemaphoreType.DMA,
        pltpu.SemaphoreType.DMA,
    ],
)


BM = 2048


def _mm2_body(x_ref, w_ref, dv_ref, o_ref):
  h = jnp.dot(x_ref[...], w_ref[...], preferred_element_type=jnp.float32)
  o_ref[0] = dv_ref[0] * h[:, :D]
  o_ref[1] = dv_ref[1] * h[:, D:]


_mm2 = pl.pallas_call(
    _mm2_body,
    grid=(NP // BM,),
    in_specs=[
        pl.BlockSpec((BM, D), lambda i: (i, 0)),
        pl.BlockSpec((D, 2 * D), lambda i: (0, 0)),
        pl.BlockSpec((2, BM, 1), lambda i: (0, i, 0)),
    ],
    out_specs=pl.BlockSpec((2, BM, D), lambda i: (0, i, 0)),
    out_shape=jax.ShapeDtypeStruct((2, NP, D), jnp.float32),
)


def _mid_body(p_ref, hs_ref, b_ref, dv_ref, w_ref, o_ref):
  t1 = jnp.maximum(
      dv_ref[0] * (p_ref[0] + hs_ref[0]) + b_ref[0], 0.0)
  o_ref[0] = dv_ref[0] * jnp.dot(
      t1, w_ref[0], preferred_element_type=jnp.float32)
  t2 = jnp.maximum(
      dv_ref[1] * (p_ref[1] + hs_ref[1]) + b_ref[1], 0.0)
  o_ref[1] = dv_ref[1] * jnp.dot(
      t2, w_ref[1], preferred_element_type=jnp.float32)


_mid = pl.pallas_call(
    _mid_body,
    grid=(NP // BM,),
    in_specs=[
        pl.BlockSpec((2, BM, D), lambda i: (0, i, 0)),
        pl.BlockSpec((2, BM, D), lambda i: (0, i, 0)),
        pl.BlockSpec((2, 1, D), lambda i: (0, 0, 0)),
        pl.BlockSpec((2, BM, 1), lambda i: (0, i, 0)),
        pl.BlockSpec((2, D, D), lambda i: (0, 0, 0)),
    ],
    out_specs=pl.BlockSpec((2, BM, D), lambda i: (0, i, 0)),
    out_shape=jax.ShapeDtypeStruct((2, NP, D), jnp.float32),
)


BF = 2000


def _fin_body(p_ref, hs_ref, b_ref, dv_ref, o_ref):
  t1 = jnp.maximum(
      dv_ref[0] * (p_ref[0] + hs_ref[0]) + b_ref[0], 0.0)
  t2 = jnp.maximum(
      dv_ref[1] * (p_ref[1] + hs_ref[1]) + b_ref[1], 0.0)
  o_ref[...] = jnp.concatenate([t1, t2], axis=1)


_fin = pl.pallas_call(
    _fin_body,
    grid=(N // BF,),
    in_specs=[
        pl.BlockSpec((2, BF, D), lambda i: (0, i, 0)),
        pl.BlockSpec((2, BF, D), lambda i: (0, i, 0)),
        pl.BlockSpec((2, 1, D), lambda i: (0, 0, 0)),
        pl.BlockSpec((2, BF, 1), lambda i: (0, i, 0)),
    ],
    out_specs=pl.BlockSpec((BF, 2 * D), lambda i: (i, 0)),
    out_shape=jax.ShapeDtypeStruct((N, 2 * D), jnp.float32),
)


def _prep_edges(ei, src_off):
  src = ei[0].astype(jnp.int32)
  dst = ei[1].astype(jnp.int32)
  pad = N + jnp.arange(EP - E, dtype=jnp.int32) % (NP - N)
  src2 = jnp.concatenate([src + src_off, pad + src_off]).reshape(NBLKS, K)
  dst2 = jnp.concatenate([dst, pad]).reshape(NBLKS, K)
  return src2, dst2


def kernel(x, edge_index, edge_index_cross, W1, b1, W2, b2,
           Wc1, bc1, Wc2, bc2):
  f32 = jnp.float32
  xp = jnp.zeros((NP, D), f32).at[:N].set(x.astype(f32))
  s1, t1 = _prep_edges(edge_index, 0)
  s2, t2 = _prep_edges(edge_index_cross, NP)
  scat = jnp.concatenate([s1, s2], axis=0)
  tcat = jnp.concatenate([t1, t2], axis=0)

  deg = _deg(tcat).reshape(2, NP) + 1.0
  mask = (jnp.arange(NP) < N).astype(f32)
  dv = (lax.rsqrt(deg) * mask)[:, :, None]

  Wcat = jnp.concatenate([W1.astype(f32), Wc1.astype(f32)], axis=1)
  hs1 = _mm2(xp, Wcat, dv)

  p1 = _agg(hs1.reshape(2 * NP, D), scat, tcat).reshape(2, NP, D)

  bcat1 = jnp.stack([b1.reshape(1, D), bc1.reshape(1, D)]).astype(f32)
  bcat2 = jnp.stack([b2.reshape(1, D), bc2.reshape(1, D)]).astype(f32)
  W2cat = jnp.stack([W2.astype(f32), Wc2.astype(f32)])

  hs2 = _mid(p1, hs1, bcat1, dv, W2cat)

  p2 = _agg(hs2.reshape(2 * NP, D), scat, tcat).reshape(2, NP, D)

  return _fin(p2, hs2, bcat2, dv)


# speedup vs baseline: 30.2337x; 1.0099x over previous
import functools

import jax
import jax.numpy as jnp
from jax import lax
from jax.experimental import pallas as pl
from jax.experimental.pallas import tpu as pltpu
from jax.experimental.pallas import tpu_sc as plsc

N = 10000
D = 128
NP = 10240
E = 320000
EP = 327680
K = 128
NBLKS = EP // K
TBLK = NBLKS // 16
CHK = 16
NCHK = TBLK // CHK
RPT = NP // 16

_MESH = plsc.VectorSubcoreMesh(core_axis_name="c", subcore_axis_name="s")


def _deg_body(dst_hbm, out_hbm, dst_sl, ones_v, zrow_v, dacc):
  c = lax.axis_index("c")
  s = lax.axis_index("s")
  w = c * 16 + s
  for j in range(K // 16):
    ones_v[pl.ds(j * 16, 16)] = jnp.ones((16,), jnp.float32)
  def zfill(i, _):
    zrow_v[pl.ds(i * 16, 16)] = jnp.zeros((16,), jnp.float32)
    return _
  lax.fori_loop(0, RPT // 16, zfill, None)
  pltpu.sync_copy(zrow_v, dacc.at[pl.ds(s * RPT, RPT)])
  pltpu.sync_copy(dst_hbm.at[pl.ds(w * TBLK, TBLK)], dst_sl)
  plsc.subcore_barrier()
  def blk(i, _):
    pltpu.sync_copy(ones_v, dacc.at[dst_sl.at[i]], add=True)
    return _
  lax.fori_loop(0, TBLK, blk, None)
  plsc.subcore_barrier()
  pltpu.sync_copy(dacc.at[pl.ds(s * RPT, RPT)], out_hbm.at[w])


_deg = pl.kernel(
    _deg_body,
    out_type=jax.ShapeDtypeStruct((32, RPT), jnp.float32),
    mesh=_MESH,
    scratch_types=[
        pltpu.VMEM((TBLK, K), jnp.int32),
        pltpu.VMEM((K,), jnp.float32),
        pltpu.VMEM((RPT,), jnp.float32),
        pltpu.VMEM_SHARED((NP,), jnp.float32),
    ],
)


def _agg_body(hs_hbm, src_hbm, dst_hbm, out_hbm,
              src_sl, dst_sl, rows_a, rows_b, acc,
              sem_a, sem_b, sem_ss, sem_sd):
  c = lax.axis_index("c")
  s = lax.axis_index("s")
  w = c * 16 + s
  base = w * TBLK
  def zfill(i, _):
    for j in range(D // 16):
      rows_a[i, pl.ds(j * 16, 16)] = jnp.zeros((16,), jnp.float32)
    return _
  lax.fori_loop(0, K, zfill, None)
  def zcp(i, _):
    pltpu.sync_copy(rows_a, acc.at[pl.ds(s * RPT + i * K, K)])
    return _
  lax.fori_loop(0, RPT // K, zcp, None)
  plsc.subcore_barrier()

  def slab_issue(ck, p):
    pltpu.async_copy(src_hbm.at[pl.ds(base + ck * CHK, CHK)],
                     src_sl.at[p], sem_ss)
    pltpu.async_copy(dst_hbm.at[pl.ds(base + ck * CHK, CHK)],
                     dst_sl.at[p], sem_sd)

  def slab_wait(ck, p):
    pltpu.make_async_copy(src_hbm.at[pl.ds(base + ck * CHK, CHK)],
                          src_sl.at[p], sem_ss).wait()
    pltpu.make_async_copy(dst_hbm.at[pl.ds(base + ck * CHK, CHK)],
                          dst_sl.at[p], sem_sd).wait()

  slab_issue(0, 0)
  def chunk(ck, _):
    p = ck % 2
    slab_wait(ck, p)
    @pl.when(ck + 1 < NCHK)
    def _issue_next():
      slab_issue(ck + 1, (ck + 1) % 2)
    pltpu.async_copy(hs_hbm.at[src_sl.at[p, 0]], rows_a, sem_a)
    def pair(j, _2):
      e = 2 * j
      pltpu.async_copy(hs_hbm.at[src_sl.at[p, e + 1]], rows_b, sem_b)
      pltpu.make_async_copy(hs_hbm.at[src_sl.at[p, e]], rows_a, sem_a).wait()
      pltpu.sync_copy(rows_a, acc.at[dst_sl.at[p, e]], add=True)
      nxt = jnp.minimum(e + 2, CHK - 1)
      pltpu.async_copy(hs_hbm.at[src_sl.at[p, nxt]], rows_a, sem_a)
      pltpu.make_async_copy(hs_hbm.at[src_sl.at[p, e + 1]], rows_b,
                            sem_b).wait()
      pltpu.sync_copy(rows_b, acc.at[dst_sl.at[p, e + 1]], add=True)
      return _2
    lax.fori_loop(0, CHK // 2, pair, None)
    pltpu.make_async_copy(hs_hbm.at[src_sl.at[p, CHK - 1]], rows_a,
                          sem_a).wait()
    return _
  lax.fori_loop(0, NCHK, chunk, None)
  plsc.subcore_barrier()
  pltpu.sync_copy(acc.at[pl.ds(s * RPT, RPT)], out_hbm.at[w])


_agg = pl.kernel(
    _agg_body,
    out_type=jax.ShapeDtypeStruct((32, RPT, D), jnp.float32),
    mesh=_MESH,
    scratch_types=[
        pltpu.VMEM((2, CHK, K), jnp.int32),
        pltpu.VMEM((2, CHK, K), jnp.int32),
        pltpu.VMEM((K, D), jnp.float32),
        pltpu.VMEM((K, D), jnp.float32),
        pltpu.VMEM_SHARED((NP, D), jnp.float32),
        pltpu.SemaphoreType.DMA,
        pltpu.SemaphoreType.DMA,
        pltpu.SemaphoreType.DMA,
        pltpu.SemaphoreType.DMA,
    ],
)


BM = 2048


def _mm2_body(x_ref, w_ref, degp_ref, rmask_ref, o_ref, dv_ref):
  dv0 = lax.rsqrt(degp_ref[0] + 1.0) * rmask_ref[...]
  dv1 = lax.rsqrt(degp_ref[1] + 1.0) * rmask_ref[...]
  dv_ref[0] = dv0
  dv_ref[1] = dv1
  h = jnp.dot(x_ref[...], w_ref[...], preferred_element_type=jnp.float32)
  o_ref[0] = dv0 * h[:, :D]
  o_ref[1] = dv1 * h[:, D:]


_mm2 = pl.pallas_call(
    _mm2_body,
    grid=(NP // BM,),
    in_specs=[
        pl.BlockSpec((BM, D), lambda i: (i, 0)),
        pl.BlockSpec((D, 2 * D), lambda i: (0, 0)),
        pl.BlockSpec((2, BM, 1), lambda i: (0, i, 0)),
        pl.BlockSpec((BM, 1), lambda i: (i, 0)),
    ],
    out_specs=[
        pl.BlockSpec((2, BM, D), lambda i: (0, i, 0)),
        pl.BlockSpec((2, BM, 1), lambda i: (0, i, 0)),
    ],
    out_shape=[
        jax.ShapeDtypeStruct((2, NP, D), jnp.float32),
        jax.ShapeDtypeStruct((2, NP, 1), jnp.float32),
    ],
)


def _mid_body(p_ref, hs_ref, b_ref, dv_ref, w_ref, o_ref):
  t1 = jnp.maximum(
      dv_ref[0] * (p_ref[0] + hs_ref[0]) + b_ref[0], 0.0)
  o_ref[0] = dv_ref[0] * jnp.dot(
      t1, w_ref[0], preferred_element_type=jnp.float32)
  t2 = jnp.maximum(
      dv_ref[1] * (p_ref[1] + hs_ref[1]) + b_ref[1], 0.0)
  o_ref[1] = dv_ref[1] * jnp.dot(
      t2, w_ref[1], preferred_element_type=jnp.float32)


_mid = pl.pallas_call(
    _mid_body,
    grid=(NP // BM,),
    in_specs=[
        pl.BlockSpec((2, BM, D), lambda i: (0, i, 0)),
        pl.BlockSpec((2, BM, D), lambda i: (0, i, 0)),
        pl.BlockSpec((2, 1, D), lambda i: (0, 0, 0)),
        pl.BlockSpec((2, BM, 1), lambda i: (0, i, 0)),
        pl.BlockSpec((2, D, D), lambda i: (0, 0, 0)),
    ],
    out_specs=pl.BlockSpec((2, BM, D), lambda i: (0, i, 0)),
    out_shape=jax.ShapeDtypeStruct((2, NP, D), jnp.float32),
)


BF = 2000


def _fin_body(p_ref, hs_ref, b_ref, dv_ref, o_ref):
  t1 = jnp.maximum(
      dv_ref[0] * (p_ref[0] + hs_ref[0]) + b_ref[0], 0.0)
  t2 = jnp.maximum(
      dv_ref[1] * (p_ref[1] + hs_ref[1]) + b_ref[1], 0.0)
  o_ref[...] = jnp.concatenate([t1, t2], axis=1)


_fin = pl.pallas_call(
    _fin_body,
    grid=(N // BF,),
    in_specs=[
        pl.BlockSpec((2, BF, D), lambda i: (0, i, 0)),
        pl.BlockSpec((2, BF, D), lambda i: (0, i, 0)),
        pl.BlockSpec((2, 1, D), lambda i: (0, 0, 0)),
        pl.BlockSpec((2, BF, 1), lambda i: (0, i, 0)),
    ],
    out_specs=pl.BlockSpec((BF, 2 * D), lambda i: (i, 0)),
    out_shape=jax.ShapeDtypeStruct((N, 2 * D), jnp.float32),
)


def _prep_edges(ei, src_off):
  src = ei[0].astype(jnp.int32)
  dst = ei[1].astype(jnp.int32)
  pad = N + jnp.arange(EP - E, dtype=jnp.int32) % (NP - N)
  src2 = jnp.concatenate([src + src_off, pad + src_off]).reshape(NBLKS, K)
  dst2 = jnp.concatenate([dst, pad]).reshape(NBLKS, K)
  return src2, dst2


def kernel(x, edge_index, edge_index_cross, W1, b1, W2, b2,
           Wc1, bc1, Wc2, bc2):
  f32 = jnp.float32
  xp = jnp.zeros((NP, D), f32).at[:N].set(x.astype(f32))
  s1, t1 = _prep_edges(edge_index, 0)
  s2, t2 = _prep_edges(edge_index_cross, NP)
  scat = jnp.concatenate([s1, s2], axis=0)
  tcat = jnp.concatenate([t1, t2], axis=0)

  degp = _deg(tcat).reshape(2, NP, 1)
  rmask = (jnp.arange(NP) < N).astype(f32).reshape(NP, 1)

  Wcat = jnp.concatenate([W1.astype(f32), Wc1.astype(f32)], axis=1)
  hs1, dv = _mm2(xp, Wcat, degp, rmask)

  p1 = _agg(hs1.reshape(2 * NP, D), scat, tcat).reshape(2, NP, D)

  bcat1 = jnp.stack([b1.reshape(1, D), bc1.reshape(1, D)]).astype(f32)
  bcat2 = jnp.stack([b2.reshape(1, D), bc2.reshape(1, D)]).astype(f32)
  W2cat = jnp.stack([W2.astype(f32), Wc2.astype(f32)])

  hs2 = _mid(p1, hs1, bcat1, dv, W2cat)

  p2 = _agg(hs2.reshape(2 * NP, D), scat, tcat).reshape(2, NP, D)

  return _fin(p2, hs2, bcat2, dv)


# speedup vs baseline: 32.5877x; 1.0779x over previous
import functools

import jax
import jax.numpy as jnp
from jax import lax
from jax.experimental import pallas as pl
from jax.experimental.pallas import tpu as pltpu
from jax.experimental.pallas import tpu_sc as plsc

N = 10000
D = 128
NP = 10240
E = 320000
EP = 327680
K = 128
NBLKS = EP // K
TBLK = NBLKS // 16
CHK = 16
NCHK = TBLK // CHK
RPT = NP // 16

_MESH = plsc.VectorSubcoreMesh(core_axis_name="c", subcore_axis_name="s")


def _deg_body(dst_hbm, out_hbm, dst_sl, ones_v, zrow_v, dacc):
  c = lax.axis_index("c")
  s = lax.axis_index("s")
  w = c * 16 + s
  for j in range(K // 16):
    ones_v[pl.ds(j * 16, 16)] = jnp.ones((16,), jnp.float32)
  def zfill(i, _):
    zrow_v[pl.ds(i * 16, 16)] = jnp.zeros((16,), jnp.float32)
    return _
  lax.fori_loop(0, RPT // 16, zfill, None)
  pltpu.sync_copy(zrow_v, dacc.at[pl.ds(s * RPT, RPT)])
  pltpu.sync_copy(dst_hbm.at[pl.ds(w * TBLK, TBLK)], dst_sl)
  plsc.subcore_barrier()
  def blk(i, _):
    pltpu.sync_copy(ones_v, dacc.at[dst_sl.at[i]], add=True)
    return _
  lax.fori_loop(0, TBLK, blk, None)
  plsc.subcore_barrier()
  pltpu.sync_copy(dacc.at[pl.ds(s * RPT, RPT)], out_hbm.at[w])


_deg = pl.kernel(
    _deg_body,
    out_type=jax.ShapeDtypeStruct((32, RPT), jnp.float32),
    mesh=_MESH,
    scratch_types=[
        pltpu.VMEM((TBLK, K), jnp.int32),
        pltpu.VMEM((K,), jnp.float32),
        pltpu.VMEM((RPT,), jnp.float32),
        pltpu.VMEM_SHARED((NP,), jnp.float32),
    ],
)


def _agg_body(hs_hbm, src_hbm, dst_hbm, out_hbm,
              src_sl, dst_sl, rows_a, rows_b, acc,
              sem_a, sem_b, sem_ss, sem_sd):
  c = lax.axis_index("c")
  s = lax.axis_index("s")
  w = c * 16 + s
  base = w * TBLK
  def zfill(i, _):
    for j in range(D // 16):
      rows_a[i, pl.ds(j * 16, 16)] = jnp.zeros((16,), jnp.float32)
    return _
  lax.fori_loop(0, K, zfill, None)
  def zcp(i, _):
    pltpu.sync_copy(rows_a, acc.at[pl.ds(s * RPT + i * K, K)])
    return _
  lax.fori_loop(0, RPT // K, zcp, None)
  plsc.subcore_barrier()

  def slab_issue(ck, p):
    pltpu.async_copy(src_hbm.at[pl.ds(base + ck * CHK, CHK)],
                     src_sl.at[p], sem_ss)
    pltpu.async_copy(dst_hbm.at[pl.ds(base + ck * CHK, CHK)],
                     dst_sl.at[p], sem_sd)

  def slab_wait(ck, p):
    pltpu.make_async_copy(src_hbm.at[pl.ds(base + ck * CHK, CHK)],
                          src_sl.at[p], sem_ss).wait()
    pltpu.make_async_copy(dst_hbm.at[pl.ds(base + ck * CHK, CHK)],
                          dst_sl.at[p], sem_sd).wait()

  slab_issue(0, 0)
  slab_wait(0, 0)
  slab_issue(1, 1)
  pltpu.async_copy(hs_hbm.at[src_sl.at[0, 0]], rows_a, sem_a)
  def chunk(ck, _):
    p = ck % 2
    pn = (ck + 1) % 2
    def pair(j, _2):
      e = 2 * j
      pltpu.async_copy(hs_hbm.at[src_sl.at[p, e + 1]], rows_b, sem_b)
      pltpu.make_async_copy(hs_hbm.at[src_sl.at[p, e]], rows_a, sem_a).wait()
      pltpu.sync_copy(rows_a, acc.at[dst_sl.at[p, e]], add=True)
      pltpu.async_copy(hs_hbm.at[src_sl.at[p, e + 2]], rows_a, sem_a)
      pltpu.make_async_copy(hs_hbm.at[src_sl.at[p, e + 1]], rows_b,
                            sem_b).wait()
      pltpu.sync_copy(rows_b, acc.at[dst_sl.at[p, e + 1]], add=True)
      return _2
    lax.fori_loop(0, CHK // 2 - 1, pair, None)
    @pl.when(ck + 1 < NCHK)
    def _wait_next_slab():
      slab_wait(ck + 1, pn)
    pltpu.async_copy(hs_hbm.at[src_sl.at[p, CHK - 1]], rows_b, sem_b)
    pltpu.make_async_copy(hs_hbm.at[src_sl.at[p, CHK - 2]], rows_a,
                          sem_a).wait()
    pltpu.sync_copy(rows_a, acc.at[dst_sl.at[p, CHK - 2]], add=True)
    @pl.when(ck + 1 < NCHK)
    def _cross_gather():
      pltpu.async_copy(hs_hbm.at[src_sl.at[pn, 0]], rows_a, sem_a)
    pltpu.make_async_copy(hs_hbm.at[src_sl.at[p, CHK - 1]], rows_b,
                          sem_b).wait()
    pltpu.sync_copy(rows_b, acc.at[dst_sl.at[p, CHK - 1]], add=True)
    @pl.when(ck + 2 < NCHK)
    def _issue_next_slab():
      slab_issue(ck + 2, p)
    return _
  lax.fori_loop(0, NCHK, chunk, None)
  plsc.subcore_barrier()
  pltpu.sync_copy(acc.at[pl.ds(s * RPT, RPT)], out_hbm.at[w])


_agg = pl.kernel(
    _agg_body,
    out_type=jax.ShapeDtypeStruct((32, RPT, D), jnp.float32),
    mesh=_MESH,
    scratch_types=[
        pltpu.VMEM((2, CHK, K), jnp.int32),
        pltpu.VMEM((2, CHK, K), jnp.int32),
        pltpu.VMEM((K, D), jnp.float32),
        pltpu.VMEM((K, D), jnp.float32),
        pltpu.VMEM_SHARED((NP, D), jnp.float32),
        pltpu.SemaphoreType.DMA,
        pltpu.SemaphoreType.DMA,
        pltpu.SemaphoreType.DMA,
        pltpu.SemaphoreType.DMA,
    ],
)


BM = 2048


def _mm2_body(x_ref, w_ref, degp_ref, rmask_ref, o_ref, dv_ref):
  dv0 = lax.rsqrt(degp_ref[0] + 1.0) * rmask_ref[...]
  dv1 = lax.rsqrt(degp_ref[1] + 1.0) * rmask_ref[...]
  dv_ref[0] = dv0
  dv_ref[1] = dv1
  h = jnp.dot(x_ref[...], w_ref[...], preferred_element_type=jnp.float32)
  o_ref[0] = dv0 * h[:, :D]
  o_ref[1] = dv1 * h[:, D:]


_mm2 = pl.pallas_call(
    _mm2_body,
    grid=(NP // BM,),
    in_specs=[
        pl.BlockSpec((BM, D), lambda i: (i, 0)),
        pl.BlockSpec((D, 2 * D), lambda i: (0, 0)),
        pl.BlockSpec((2, BM, 1), lambda i: (0, i, 0)),
        pl.BlockSpec((BM, 1), lambda i: (i, 0)),
    ],
    out_specs=[
        pl.BlockSpec((2, BM, D), lambda i: (0, i, 0)),
        pl.BlockSpec((2, BM, 1), lambda i: (0, i, 0)),
    ],
    out_shape=[
        jax.ShapeDtypeStruct((2, NP, D), jnp.float32),
        jax.ShapeDtypeStruct((2, NP, 1), jnp.float32),
    ],
)


def _mid_body(p_ref, hs_ref, b_ref, dv_ref, w_ref, o_ref):
  t1 = jnp.maximum(
      dv_ref[0] * (p_ref[0] + hs_ref[0]) + b_ref[0], 0.0)
  o_ref[0] = dv_ref[0] * jnp.dot(
      t1, w_ref[0], preferred_element_type=jnp.float32)
  t2 = jnp.maximum(
      dv_ref[1] * (p_ref[1] + hs_ref[1]) + b_ref[1], 0.0)
  o_ref[1] = dv_ref[1] * jnp.dot(
      t2, w_ref[1], preferred_element_type=jnp.float32)


_mid = pl.pallas_call(
    _mid_body,
    grid=(NP // BM,),
    in_specs=[
        pl.BlockSpec((2, BM, D), lambda i: (0, i, 0)),
        pl.BlockSpec((2, BM, D), lambda i: (0, i, 0)),
        pl.BlockSpec((2, 1, D), lambda i: (0, 0, 0)),
        pl.BlockSpec((2, BM, 1), lambda i: (0, i, 0)),
        pl.BlockSpec((2, D, D), lambda i: (0, 0, 0)),
    ],
    out_specs=pl.BlockSpec((2, BM, D), lambda i: (0, i, 0)),
    out_shape=jax.ShapeDtypeStruct((2, NP, D), jnp.float32),
)


BF = 2000


def _fin_body(p_ref, hs_ref, b_ref, dv_ref, o_ref):
  t1 = jnp.maximum(
      dv_ref[0] * (p_ref[0] + hs_ref[0]) + b_ref[0], 0.0)
  t2 = jnp.maximum(
      dv_ref[1] * (p_ref[1] + hs_ref[1]) + b_ref[1], 0.0)
  o_ref[...] = jnp.concatenate([t1, t2], axis=1)


_fin = pl.pallas_call(
    _fin_body,
    grid=(N // BF,),
    in_specs=[
        pl.BlockSpec((2, BF, D), lambda i: (0, i, 0)),
        pl.BlockSpec((2, BF, D), lambda i: (0, i, 0)),
        pl.BlockSpec((2, 1, D), lambda i: (0, 0, 0)),
        pl.BlockSpec((2, BF, 1), lambda i: (0, i, 0)),
    ],
    out_specs=pl.BlockSpec((BF, 2 * D), lambda i: (i, 0)),
    out_shape=jax.ShapeDtypeStruct((N, 2 * D), jnp.float32),
)


def _prep_edges(ei, src_off):
  src = ei[0].astype(jnp.int32)
  dst = ei[1].astype(jnp.int32)
  pad = N + jnp.arange(EP - E, dtype=jnp.int32) % (NP - N)
  src2 = jnp.concatenate([src + src_off, pad + src_off]).reshape(NBLKS, K)
  dst2 = jnp.concatenate([dst, pad]).reshape(NBLKS, K)
  return src2, dst2


def kernel(x, edge_index, edge_index_cross, W1, b1, W2, b2,
           Wc1, bc1, Wc2, bc2):
  f32 = jnp.float32
  xp = jnp.zeros((NP, D), f32).at[:N].set(x.astype(f32))
  s1, t1 = _prep_edges(edge_index, 0)
  s2, t2 = _prep_edges(edge_index_cross, NP)
  scat = jnp.concatenate([s1, s2], axis=0)
  tcat = jnp.concatenate([t1, t2], axis=0)

  degp = _deg(tcat).reshape(2, NP, 1)
  rmask = (jnp.arange(NP) < N).astype(f32).reshape(NP, 1)

  Wcat = jnp.concatenate([W1.astype(f32), Wc1.astype(f32)], axis=1)
  hs1, dv = _mm2(xp, Wcat, degp, rmask)

  p1 = _agg(hs1.reshape(2 * NP, D), scat, tcat).reshape(2, NP, D)

  bcat1 = jnp.stack([b1.reshape(1, D), bc1.reshape(1, D)]).astype(f32)
  bcat2 = jnp.stack([b2.reshape(1, D), bc2.reshape(1, D)]).astype(f32)
  W2cat = jnp.stack([W2.astype(f32), Wc2.astype(f32)])

  hs2 = _mid(p1, hs1, bcat1, dv, W2cat)

  p2 = _agg(hs2.reshape(2 * NP, D), scat, tcat).reshape(2, NP, D)

  return _fin(p2, hs2, bcat2, dv)


# speedup vs baseline: 32.8157x; 1.0070x over previous
import functools

import jax
import jax.numpy as jnp
from jax import lax
from jax.experimental import pallas as pl
from jax.experimental.pallas import tpu as pltpu
from jax.experimental.pallas import tpu_sc as plsc

N = 10000
D = 128
NP = 10240
E = 320000
EP = 327680
K = 128
NBLKS = EP // K
TBLK = NBLKS // 16
CHK = 16
NCHK = TBLK // CHK
RPT = NP // 16

_MESH = plsc.VectorSubcoreMesh(core_axis_name="c", subcore_axis_name="s")


def _deg_body(dst_hbm, out_hbm, dst_sl, ones_v, zrow_v, dacc, sem_d):
  c = lax.axis_index("c")
  s = lax.axis_index("s")
  w = c * 16 + s
  for j in range(K // 16):
    ones_v[pl.ds(j * 16, 16)] = jnp.ones((16,), jnp.float32)
  def zfill(i, _):
    zrow_v[pl.ds(i * 16, 16)] = jnp.zeros((16,), jnp.float32)
    return _
  lax.fori_loop(0, RPT // 16, zfill, None)
  pltpu.sync_copy(zrow_v, dacc.at[pl.ds(s * RPT, RPT)])
  pltpu.sync_copy(dst_hbm.at[pl.ds(w * TBLK, TBLK)], dst_sl)
  plsc.subcore_barrier()
  def blk4(g, _):
    for q in range(4):
      pltpu.async_copy(ones_v, dacc.at[dst_sl.at[4 * g + q]], sem_d,
                       add=True)
    for q in range(4):
      pltpu.make_async_copy(ones_v, dacc.at[dst_sl.at[4 * g + q]],
                            sem_d).wait()
    return _
  lax.fori_loop(0, TBLK // 4, blk4, None)
  plsc.subcore_barrier()
  pltpu.sync_copy(dacc.at[pl.ds(s * RPT, RPT)], out_hbm.at[w])


_deg = pl.kernel(
    _deg_body,
    out_type=jax.ShapeDtypeStruct((32, RPT), jnp.float32),
    mesh=_MESH,
    scratch_types=[
        pltpu.VMEM((TBLK, K), jnp.int32),
        pltpu.VMEM((K,), jnp.float32),
        pltpu.VMEM((RPT,), jnp.float32),
        pltpu.VMEM_SHARED((NP,), jnp.float32),
        pltpu.SemaphoreType.DMA,
    ],
)


def _agg_body(hs_hbm, src_hbm, dst_hbm, out_hbm,
              src_sl, dst_sl, rows_a, rows_b, acc,
              sem_a, sem_b, sem_ss, sem_sd):
  c = lax.axis_index("c")
  s = lax.axis_index("s")
  w = c * 16 + s
  base = w * TBLK
  def zfill(i, _):
    for j in range(D // 16):
      rows_a[i, pl.ds(j * 16, 16)] = jnp.zeros((16,), jnp.float32)
    return _
  lax.fori_loop(0, K, zfill, None)
  def zcp(i, _):
    pltpu.sync_copy(rows_a, acc.at[pl.ds(s * RPT + i * K, K)])
    return _
  lax.fori_loop(0, RPT // K, zcp, None)
  plsc.subcore_barrier()

  def slab_issue(ck, p):
    pltpu.async_copy(src_hbm.at[pl.ds(base + ck * CHK, CHK)],
                     src_sl.at[p], sem_ss)
    pltpu.async_copy(dst_hbm.at[pl.ds(base + ck * CHK, CHK)],
                     dst_sl.at[p], sem_sd)

  def slab_wait(ck, p):
    pltpu.make_async_copy(src_hbm.at[pl.ds(base + ck * CHK, CHK)],
                          src_sl.at[p], sem_ss).wait()
    pltpu.make_async_copy(dst_hbm.at[pl.ds(base + ck * CHK, CHK)],
                          dst_sl.at[p], sem_sd).wait()

  slab_issue(0, 0)
  slab_wait(0, 0)
  slab_issue(1, 1)
  pltpu.async_copy(hs_hbm.at[src_sl.at[0, 0]], rows_a, sem_a)
  def chunk(ck, _):
    p = ck % 2
    pn = (ck + 1) % 2
    def pair(j, _2):
      e = 2 * j
      pltpu.async_copy(hs_hbm.at[src_sl.at[p, e + 1]], rows_b, sem_b)
      pltpu.make_async_copy(hs_hbm.at[src_sl.at[p, e]], rows_a, sem_a).wait()
      pltpu.sync_copy(rows_a, acc.at[dst_sl.at[p, e]], add=True)
      pltpu.async_copy(hs_hbm.at[src_sl.at[p, e + 2]], rows_a, sem_a)
      pltpu.make_async_copy(hs_hbm.at[src_sl.at[p, e + 1]], rows_b,
                            sem_b).wait()
      pltpu.sync_copy(rows_b, acc.at[dst_sl.at[p, e + 1]], add=True)
      return _2
    lax.fori_loop(0, CHK // 2 - 1, pair, None)
    @pl.when(ck + 1 < NCHK)
    def _wait_next_slab():
      slab_wait(ck + 1, pn)
    pltpu.async_copy(hs_hbm.at[src_sl.at[p, CHK - 1]], rows_b, sem_b)
    pltpu.make_async_copy(hs_hbm.at[src_sl.at[p, CHK - 2]], rows_a,
                          sem_a).wait()
    pltpu.sync_copy(rows_a, acc.at[dst_sl.at[p, CHK - 2]], add=True)
    @pl.when(ck + 1 < NCHK)
    def _cross_gather():
      pltpu.async_copy(hs_hbm.at[src_sl.at[pn, 0]], rows_a, sem_a)
    pltpu.make_async_copy(hs_hbm.at[src_sl.at[p, CHK - 1]], rows_b,
                          sem_b).wait()
    pltpu.sync_copy(rows_b, acc.at[dst_sl.at[p, CHK - 1]], add=True)
    @pl.when(ck + 2 < NCHK)
    def _issue_next_slab():
      slab_issue(ck + 2, p)
    return _
  lax.fori_loop(0, NCHK, chunk, None)
  plsc.subcore_barrier()
  pltpu.sync_copy(acc.at[pl.ds(s * RPT, RPT)], out_hbm.at[w])


_agg = pl.kernel(
    _agg_body,
    out_type=jax.ShapeDtypeStruct((32, RPT, D), jnp.float32),
    mesh=_MESH,
    scratch_types=[
        pltpu.VMEM((2, CHK, K), jnp.int32),
        pltpu.VMEM((2, CHK, K), jnp.int32),
        pltpu.VMEM((K, D), jnp.float32),
        pltpu.VMEM((K, D), jnp.float32),
        pltpu.VMEM_SHARED((NP, D), jnp.float32),
        pltpu.SemaphoreType.DMA,
        pltpu.SemaphoreType.DMA,
        pltpu.SemaphoreType.DMA,
        pltpu.SemaphoreType.DMA,
    ],
)


BM = 2048


def _mm2_body(x_ref, w_ref, degp_ref, rmask_ref, o_ref, dv_ref):
  dv0 = lax.rsqrt(degp_ref[0] + 1.0) * rmask_ref[...]
  dv1 = lax.rsqrt(degp_ref[1] + 1.0) * rmask_ref[...]
  dv_ref[0] = dv0
  dv_ref[1] = dv1
  h = jnp.dot(x_ref[...], w_ref[...], preferred_element_type=jnp.float32)
  o_ref[0] = dv0 * h[:, :D]
  o_ref[1] = dv1 * h[:, D:]


_mm2 = pl.pallas_call(
    _mm2_body,
    grid=(NP // BM,),
    in_specs=[
        pl.BlockSpec((BM, D), lambda i: (i, 0)),
        pl.BlockSpec((D, 2 * D), lambda i: (0, 0)),
        pl.BlockSpec((2, BM, 1), lambda i: (0, i, 0)),
        pl.BlockSpec((BM, 1), lambda i: (i, 0)),
    ],
    out_specs=[
        pl.BlockSpec((2, BM, D), lambda i: (0, i, 0)),
        pl.BlockSpec((2, BM, 1), lambda i: (0, i, 0)),
    ],
    out_shape=[
        jax.ShapeDtypeStruct((2, NP, D), jnp.float32),
        jax.ShapeDtypeStruct((2, NP, 1), jnp.float32),
    ],
)


def _mid_body(p_ref, hs_ref, b_ref, dv_ref, w_ref, o_ref):
  t1 = jnp.maximum(
      dv_ref[0] * (p_ref[0] + hs_ref[0]) + b_ref[0], 0.0)
  o_ref[0] = dv_ref[0] * jnp.dot(
      t1, w_ref[0], preferred_element_type=jnp.float32)
  t2 = jnp.maximum(
      dv_ref[1] * (p_ref[1] + hs_ref[1]) + b_ref[1], 0.0)
  o_ref[1] = dv_ref[1] * jnp.dot(
      t2, w_ref[1], preferred_element_type=jnp.float32)


_mid = pl.pallas_call(
    _mid_body,
    grid=(NP // BM,),
    in_specs=[
        pl.BlockSpec((2, BM, D), lambda i: (0, i, 0)),
        pl.BlockSpec((2, BM, D), lambda i: (0, i, 0)),
        pl.BlockSpec((2, 1, D), lambda i: (0, 0, 0)),
        pl.BlockSpec((2, BM, 1), lambda i: (0, i, 0)),
        pl.BlockSpec((2, D, D), lambda i: (0, 0, 0)),
    ],
    out_specs=pl.BlockSpec((2, BM, D), lambda i: (0, i, 0)),
    out_shape=jax.ShapeDtypeStruct((2, NP, D), jnp.float32),
)


BF = 2000


def _fin_body(p_ref, hs_ref, b_ref, dv_ref, o_ref):
  t1 = jnp.maximum(
      dv_ref[0] * (p_ref[0] + hs_ref[0]) + b_ref[0], 0.0)
  t2 = jnp.maximum(
      dv_ref[1] * (p_ref[1] + hs_ref[1]) + b_ref[1], 0.0)
  o_ref[...] = jnp.concatenate([t1, t2], axis=1)


_fin = pl.pallas_call(
    _fin_body,
    grid=(N // BF,),
    in_specs=[
        pl.BlockSpec((2, BF, D), lambda i: (0, i, 0)),
        pl.BlockSpec((2, BF, D), lambda i: (0, i, 0)),
        pl.BlockSpec((2, 1, D), lambda i: (0, 0, 0)),
        pl.BlockSpec((2, BF, 1), lambda i: (0, i, 0)),
    ],
    out_specs=pl.BlockSpec((BF, 2 * D), lambda i: (i, 0)),
    out_shape=jax.ShapeDtypeStruct((N, 2 * D), jnp.float32),
)


def _prep_edges(ei, src_off):
  src = ei[0].astype(jnp.int32)
  dst = ei[1].astype(jnp.int32)
  pad = N + jnp.arange(EP - E, dtype=jnp.int32) % (NP - N)
  src2 = jnp.concatenate([src + src_off, pad + src_off]).reshape(NBLKS, K)
  dst2 = jnp.concatenate([dst, pad]).reshape(NBLKS, K)
  return src2, dst2


def kernel(x, edge_index, edge_index_cross, W1, b1, W2, b2,
           Wc1, bc1, Wc2, bc2):
  f32 = jnp.float32
  xp = jnp.zeros((NP, D), f32).at[:N].set(x.astype(f32))
  s1, t1 = _prep_edges(edge_index, 0)
  s2, t2 = _prep_edges(edge_index_cross, NP)
  scat = jnp.concatenate([s1, s2], axis=0)
  tcat = jnp.concatenate([t1, t2], axis=0)

  degp = _deg(tcat).reshape(2, NP, 1)
  rmask = (jnp.arange(NP) < N).astype(f32).reshape(NP, 1)

  Wcat = jnp.concatenate([W1.astype(f32), Wc1.astype(f32)], axis=1)
  hs1, dv = _mm2(xp, Wcat, degp, rmask)

  p1 = _agg(hs1.reshape(2 * NP, D), scat, tcat).reshape(2, NP, D)

  bcat1 = jnp.stack([b1.reshape(1, D), bc1.reshape(1, D)]).astype(f32)
  bcat2 = jnp.stack([b2.reshape(1, D), bc2.reshape(1, D)]).astype(f32)
  W2cat = jnp.stack([W2.astype(f32), Wc2.astype(f32)])

  hs2 = _mid(p1, hs1, bcat1, dv, W2cat)

  p2 = _agg(hs2.reshape(2 * NP, D), scat, tcat).reshape(2, NP, D)

  return _fin(p2, hs2, bcat2, dv)


# speedup vs baseline: 32.9073x; 1.0028x over previous
import functools

import jax
import jax.numpy as jnp
from jax import lax
from jax.experimental import pallas as pl
from jax.experimental.pallas import tpu as pltpu
from jax.experimental.pallas import tpu_sc as plsc

N = 10000
D = 128
NP = 10240
E = 320000
EP = 327680
K = 128
NBLKS = EP // K
TBLK = NBLKS // 16
CHK = 16
NCHK = TBLK // CHK
RPT = NP // 16

_MESH = plsc.VectorSubcoreMesh(core_axis_name="c", subcore_axis_name="s")


def _deg_body(dst_hbm, out_hbm, dst_sl, ones_v, zrow_v, dacc, sem_d):
  c = lax.axis_index("c")
  s = lax.axis_index("s")
  w = c * 16 + s
  for j in range(K // 16):
    ones_v[pl.ds(j * 16, 16)] = jnp.ones((16,), jnp.float32)
  def zfill(i, _):
    zrow_v[pl.ds(i * 16, 16)] = jnp.zeros((16,), jnp.float32)
    return _
  lax.fori_loop(0, RPT // 16, zfill, None)
  pltpu.sync_copy(zrow_v, dacc.at[pl.ds(s * RPT, RPT)])
  pltpu.sync_copy(dst_hbm.at[pl.ds(w * TBLK, TBLK)], dst_sl)
  plsc.subcore_barrier()
  def blk4(g, _):
    for q in range(4):
      pltpu.async_copy(ones_v, dacc.at[dst_sl.at[4 * g + q]], sem_d,
                       add=True)
    for q in range(4):
      pltpu.make_async_copy(ones_v, dacc.at[dst_sl.at[4 * g + q]],
                            sem_d).wait()
    return _
  lax.fori_loop(0, TBLK // 4, blk4, None)
  plsc.subcore_barrier()
  pltpu.sync_copy(dacc.at[pl.ds(s * RPT, RPT)], out_hbm.at[w])


_deg = pl.kernel(
    _deg_body,
    out_type=jax.ShapeDtypeStruct((32, RPT), jnp.float32),
    mesh=_MESH,
    scratch_types=[
        pltpu.VMEM((TBLK, K), jnp.int32),
        pltpu.VMEM((K,), jnp.float32),
        pltpu.VMEM((RPT,), jnp.float32),
        pltpu.VMEM_SHARED((NP,), jnp.float32),
        pltpu.SemaphoreType.DMA,
    ],
)


def _agg_body(hs_hbm, src_hbm, dst_hbm, out_hbm,
              src_sl, dst_sl, rows_a, rows_b, acc,
              sem_a, sem_b, sem_ss, sem_sd):
  c = lax.axis_index("c")
  s = lax.axis_index("s")
  w = c * 16 + s
  base = w * TBLK
  def zfill(i, _):
    for j in range(D // 16):
      rows_a[i, pl.ds(j * 16, 16)] = jnp.zeros((16,), jnp.float32)
    return _
  lax.fori_loop(0, K, zfill, None)
  def zcp(i, _):
    pltpu.async_copy(rows_a, acc.at[pl.ds(s * RPT + i * K, K)], sem_b)
    return _
  lax.fori_loop(0, RPT // K, zcp, None)
  def zwait(i, _):
    pltpu.make_async_copy(rows_a, acc.at[pl.ds(s * RPT + i * K, K)],
                          sem_b).wait()
    return _
  lax.fori_loop(0, RPT // K, zwait, None)
  plsc.subcore_barrier()

  def slab_issue(ck, p):
    pltpu.async_copy(src_hbm.at[pl.ds(base + ck * CHK, CHK)],
                     src_sl.at[p], sem_ss)
    pltpu.async_copy(dst_hbm.at[pl.ds(base + ck * CHK, CHK)],
                     dst_sl.at[p], sem_sd)

  def slab_wait(ck, p):
    pltpu.make_async_copy(src_hbm.at[pl.ds(base + ck * CHK, CHK)],
                          src_sl.at[p], sem_ss).wait()
    pltpu.make_async_copy(dst_hbm.at[pl.ds(base + ck * CHK, CHK)],
                          dst_sl.at[p], sem_sd).wait()

  slab_issue(0, 0)
  slab_wait(0, 0)
  slab_issue(1, 1)
  pltpu.async_copy(hs_hbm.at[src_sl.at[0, 0]], rows_a, sem_a)
  def chunk(ck, _):
    p = ck % 2
    pn = (ck + 1) % 2
    def pair(j, _2):
      e = 2 * j
      pltpu.async_copy(hs_hbm.at[src_sl.at[p, e + 1]], rows_b, sem_b)
      pltpu.make_async_copy(hs_hbm.at[src_sl.at[p, e]], rows_a, sem_a).wait()
      pltpu.sync_copy(rows_a, acc.at[dst_sl.at[p, e]], add=True)
      pltpu.async_copy(hs_hbm.at[src_sl.at[p, e + 2]], rows_a, sem_a)
      pltpu.make_async_copy(hs_hbm.at[src_sl.at[p, e + 1]], rows_b,
                            sem_b).wait()
      pltpu.sync_copy(rows_b, acc.at[dst_sl.at[p, e + 1]], add=True)
      return _2
    lax.fori_loop(0, CHK // 2 - 1, pair, None)
    @pl.when(ck + 1 < NCHK)
    def _wait_next_slab():
      slab_wait(ck + 1, pn)
    pltpu.async_copy(hs_hbm.at[src_sl.at[p, CHK - 1]], rows_b, sem_b)
    pltpu.make_async_copy(hs_hbm.at[src_sl.at[p, CHK - 2]], rows_a,
                          sem_a).wait()
    pltpu.sync_copy(rows_a, acc.at[dst_sl.at[p, CHK - 2]], add=True)
    @pl.when(ck + 1 < NCHK)
    def _cross_gather():
      pltpu.async_copy(hs_hbm.at[src_sl.at[pn, 0]], rows_a, sem_a)
    pltpu.make_async_copy(hs_hbm.at[src_sl.at[p, CHK - 1]], rows_b,
                          sem_b).wait()
    pltpu.sync_copy(rows_b, acc.at[dst_sl.at[p, CHK - 1]], add=True)
    @pl.when(ck + 2 < NCHK)
    def _issue_next_slab():
      slab_issue(ck + 2, p)
    return _
  lax.fori_loop(0, NCHK, chunk, None)
  plsc.subcore_barrier()
  pltpu.sync_copy(acc.at[pl.ds(s * RPT, RPT)], out_hbm.at[w])


_agg = pl.kernel(
    _agg_body,
    out_type=jax.ShapeDtypeStruct((32, RPT, D), jnp.float32),
    mesh=_MESH,
    scratch_types=[
        pltpu.VMEM((2, CHK, K), jnp.int32),
        pltpu.VMEM((2, CHK, K), jnp.int32),
        pltpu.VMEM((K, D), jnp.float32),
        pltpu.VMEM((K, D), jnp.float32),
        pltpu.VMEM_SHARED((NP, D), jnp.float32),
        pltpu.SemaphoreType.DMA,
        pltpu.SemaphoreType.DMA,
        pltpu.SemaphoreType.DMA,
        pltpu.SemaphoreType.DMA,
    ],
)


BM = 2048


def _mm2_body(x_ref, w_ref, degp_ref, rmask_ref, o_ref, dv_ref):
  dv0 = lax.rsqrt(degp_ref[0] + 1.0) * rmask_ref[...]
  dv1 = lax.rsqrt(degp_ref[1] + 1.0) * rmask_ref[...]
  dv_ref[0] = dv0
  dv_ref[1] = dv1
  h = jnp.dot(x_ref[...], w_ref[...], preferred_element_type=jnp.float32)
  o_ref[0] = dv0 * h[:, :D]
  o_ref[1] = dv1 * h[:, D:]


_mm2 = pl.pallas_call(
    _mm2_body,
    grid=(NP // BM,),
    in_specs=[
        pl.BlockSpec((BM, D), lambda i: (i, 0)),
        pl.BlockSpec((D, 2 * D), lambda i: (0, 0)),
        pl.BlockSpec((2, BM, 1), lambda i: (0, i, 0)),
        pl.BlockSpec((BM, 1), lambda i: (i, 0)),
    ],
    out_specs=[
        pl.BlockSpec((2, BM, D), lambda i: (0, i, 0)),
        pl.BlockSpec((2, BM, 1), lambda i: (0, i, 0)),
    ],
    out_shape=[
        jax.ShapeDtypeStruct((2, NP, D), jnp.float32),
        jax.ShapeDtypeStruct((2, NP, 1), jnp.float32),
    ],
)


def _mid_body(p_ref, hs_ref, b_ref, dv_ref, w_ref, o_ref):
  t1 = jnp.maximum(
      dv_ref[0] * (p_ref[0] + hs_ref[0]) + b_ref[0], 0.0)
  o_ref[0] = dv_ref[0] * jnp.dot(
      t1, w_ref[0], preferred_element_type=jnp.float32)
  t2 = jnp.maximum(
      dv_ref[1] * (p_ref[1] + hs_ref[1]) + b_ref[1], 0.0)
  o_ref[1] = dv_ref[1] * jnp.dot(
      t2, w_ref[1], preferred_element_type=jnp.float32)


_mid = pl.pallas_call(
    _mid_body,
    grid=(NP // BM,),
    in_specs=[
        pl.BlockSpec((2, BM, D), lambda i: (0, i, 0)),
        pl.BlockSpec((2, BM, D), lambda i: (0, i, 0)),
        pl.BlockSpec((2, 1, D), lambda i: (0, 0, 0)),
        pl.BlockSpec((2, BM, 1), lambda i: (0, i, 0)),
        pl.BlockSpec((2, D, D), lambda i: (0, 0, 0)),
    ],
    out_specs=pl.BlockSpec((2, BM, D), lambda i: (0, i, 0)),
    out_shape=jax.ShapeDtypeStruct((2, NP, D), jnp.float32),
)


BF = 2000


def _fin_body(p_ref, hs_ref, b_ref, dv_ref, o_ref):
  t1 = jnp.maximum(
      dv_ref[0] * (p_ref[0] + hs_ref[0]) + b_ref[0], 0.0)
  t2 = jnp.maximum(
      dv_ref[1] * (p_ref[1] + hs_ref[1]) + b_ref[1], 0.0)
  o_ref[...] = jnp.concatenate([t1, t2], axis=1)


_fin = pl.pallas_call(
    _fin_body,
    grid=(N // BF,),
    in_specs=[
        pl.BlockSpec((2, BF, D), lambda i: (0, i, 0)),
        pl.BlockSpec((2, BF, D), lambda i: (0, i, 0)),
        pl.BlockSpec((2, 1, D), lambda i: (0, 0, 0)),
        pl.BlockSpec((2, BF, 1), lambda i: (0, i, 0)),
    ],
    out_specs=pl.BlockSpec((BF, 2 * D), lambda i: (i, 0)),
    out_shape=jax.ShapeDtypeStruct((N, 2 * D), jnp.float32),
)


def _prep_edges(ei, src_off):
  src = ei[0].astype(jnp.int32)
  dst = ei[1].astype(jnp.int32)
  pad = N + jnp.arange(EP - E, dtype=jnp.int32) % (NP - N)
  src2 = jnp.concatenate([src + src_off, pad + src_off]).reshape(NBLKS, K)
  dst2 = jnp.concatenate([dst, pad]).reshape(NBLKS, K)
  return src2, dst2


def kernel(x, edge_index, edge_index_cross, W1, b1, W2, b2,
           Wc1, bc1, Wc2, bc2):
  f32 = jnp.float32
  xp = jnp.zeros((NP, D), f32).at[:N].set(x.astype(f32))
  s1, t1 = _prep_edges(edge_index, 0)
  s2, t2 = _prep_edges(edge_index_cross, NP)
  scat = jnp.concatenate([s1, s2], axis=0)
  tcat = jnp.concatenate([t1, t2], axis=0)

  degp = _deg(tcat).reshape(2, NP, 1)
  rmask = (jnp.arange(NP) < N).astype(f32).reshape(NP, 1)

  Wcat = jnp.concatenate([W1.astype(f32), Wc1.astype(f32)], axis=1)
  hs1, dv = _mm2(xp, Wcat, degp, rmask)

  p1 = _agg(hs1.reshape(2 * NP, D), scat, tcat).reshape(2, NP, D)

  bcat1 = jnp.stack([b1.reshape(1, D), bc1.reshape(1, D)]).astype(f32)
  bcat2 = jnp.stack([b2.reshape(1, D), bc2.reshape(1, D)]).astype(f32)
  W2cat = jnp.stack([W2.astype(f32), Wc2.astype(f32)])

  hs2 = _mid(p1, hs1, bcat1, dv, W2cat)

  p2 = _agg(hs2.reshape(2 * NP, D), scat, tcat).reshape(2, NP, D)

  return _fin(p2, hs2, bcat2, dv)


# speedup vs baseline: 33.0873x; 1.0055x over previous
import functools

import jax
import jax.numpy as jnp
from jax import lax
from jax.experimental import pallas as pl
from jax.experimental.pallas import tpu as pltpu
from jax.experimental.pallas import tpu_sc as plsc

N = 10000
D = 128
NP = 10240
E = 320000
EP = 327680
K = 128
NBLKS = EP // K
TBLK = NBLKS // 16
CHK = 16
NCHK = TBLK // CHK
RPT = NP // 16

_MESH = plsc.VectorSubcoreMesh(core_axis_name="c", subcore_axis_name="s")


def _deg_body(dst_hbm, out_hbm, dst_sl, ones_v, zrow_v, dacc, sem_d):
  c = lax.axis_index("c")
  s = lax.axis_index("s")
  w = c * 16 + s
  for j in range(K // 16):
    ones_v[pl.ds(j * 16, 16)] = jnp.ones((16,), jnp.float32)
  def zfill(i, _):
    zrow_v[pl.ds(i * 16, 16)] = jnp.zeros((16,), jnp.float32)
    return _
  lax.fori_loop(0, RPT // 16, zfill, None)
  pltpu.sync_copy(zrow_v, dacc.at[pl.ds(s * RPT, RPT)])
  pltpu.sync_copy(dst_hbm.at[pl.ds(w * TBLK, TBLK)], dst_sl)
  plsc.subcore_barrier()
  def blk4(g, _):
    for q in range(4):
      pltpu.async_copy(ones_v, dacc.at[dst_sl.at[4 * g + q]], sem_d,
                       add=True)
    for q in range(4):
      pltpu.make_async_copy(ones_v, dacc.at[dst_sl.at[4 * g + q]],
                            sem_d).wait()
    return _
  lax.fori_loop(0, TBLK // 4, blk4, None)
  plsc.subcore_barrier()
  pltpu.sync_copy(dacc.at[pl.ds(s * RPT, RPT)], out_hbm.at[w])


_deg = pl.kernel(
    _deg_body,
    out_type=jax.ShapeDtypeStruct((32, RPT), jnp.float32),
    mesh=_MESH,
    scratch_types=[
        pltpu.VMEM((TBLK, K), jnp.int32),
        pltpu.VMEM((K,), jnp.float32),
        pltpu.VMEM((RPT,), jnp.float32),
        pltpu.VMEM_SHARED((NP,), jnp.float32),
        pltpu.SemaphoreType.DMA,
    ],
)


def _agg_body(hs_hbm, src_hbm, dst_hbm, out_hbm,
              src_sl, dst_sl, rows_a, rows_b, acc,
              sem_a, sem_b, sem_ss, sem_sd):
  c = lax.axis_index("c")
  s = lax.axis_index("s")
  w = c * 16 + s
  base = w * TBLK

  def slab_issue(ck, p):
    pltpu.async_copy(src_hbm.at[pl.ds(base + ck * CHK, CHK)],
                     src_sl.at[p], sem_ss)
    pltpu.async_copy(dst_hbm.at[pl.ds(base + ck * CHK, CHK)],
                     dst_sl.at[p], sem_sd)

  def slab_wait(ck, p):
    pltpu.make_async_copy(src_hbm.at[pl.ds(base + ck * CHK, CHK)],
                          src_sl.at[p], sem_ss).wait()
    pltpu.make_async_copy(dst_hbm.at[pl.ds(base + ck * CHK, CHK)],
                          dst_sl.at[p], sem_sd).wait()

  slab_issue(0, 0)
  slab_issue(1, 1)
  def zfill(i, _):
    for j in range(D // 16):
      rows_a[i, pl.ds(j * 16, 16)] = jnp.zeros((16,), jnp.float32)
    return _
  lax.fori_loop(0, K, zfill, None)
  def zcp(i, _):
    pltpu.async_copy(rows_a, acc.at[pl.ds(s * RPT + i * K, K)], sem_b)
    return _
  lax.fori_loop(0, RPT // K, zcp, None)
  def zwait(i, _):
    pltpu.make_async_copy(rows_a, acc.at[pl.ds(s * RPT + i * K, K)],
                          sem_b).wait()
    return _
  lax.fori_loop(0, RPT // K, zwait, None)
  plsc.subcore_barrier()

  slab_wait(0, 0)
  pltpu.async_copy(hs_hbm.at[src_sl.at[0, 0]], rows_a, sem_a)
  def chunk(ck, _):
    p = ck % 2
    pn = (ck + 1) % 2
    def pair(j, _2):
      e = 2 * j
      pltpu.async_copy(hs_hbm.at[src_sl.at[p, e + 1]], rows_b, sem_b)
      pltpu.make_async_copy(hs_hbm.at[src_sl.at[p, e]], rows_a, sem_a).wait()
      pltpu.sync_copy(rows_a, acc.at[dst_sl.at[p, e]], add=True)
      pltpu.async_copy(hs_hbm.at[src_sl.at[p, e + 2]], rows_a, sem_a)
      pltpu.make_async_copy(hs_hbm.at[src_sl.at[p, e + 1]], rows_b,
                            sem_b).wait()
      pltpu.sync_copy(rows_b, acc.at[dst_sl.at[p, e + 1]], add=True)
      return _2
    lax.fori_loop(0, CHK // 2 - 1, pair, None)
    @pl.when(ck + 1 < NCHK)
    def _wait_next_slab():
      slab_wait(ck + 1, pn)
    pltpu.async_copy(hs_hbm.at[src_sl.at[p, CHK - 1]], rows_b, sem_b)
    pltpu.make_async_copy(hs_hbm.at[src_sl.at[p, CHK - 2]], rows_a,
                          sem_a).wait()
    pltpu.sync_copy(rows_a, acc.at[dst_sl.at[p, CHK - 2]], add=True)
    @pl.when(ck + 1 < NCHK)
    def _cross_gather():
      pltpu.async_copy(hs_hbm.at[src_sl.at[pn, 0]], rows_a, sem_a)
    pltpu.make_async_copy(hs_hbm.at[src_sl.at[p, CHK - 1]], rows_b,
                          sem_b).wait()
    pltpu.sync_copy(rows_b, acc.at[dst_sl.at[p, CHK - 1]], add=True)
    @pl.when(ck + 2 < NCHK)
    def _issue_next_slab():
      slab_issue(ck + 2, p)
    return _
  lax.fori_loop(0, NCHK, chunk, None)
  plsc.subcore_barrier()
  pltpu.sync_copy(acc.at[pl.ds(s * RPT, RPT)], out_hbm.at[w])


_agg = pl.kernel(
    _agg_body,
    out_type=jax.ShapeDtypeStruct((32, RPT, D), jnp.float32),
    mesh=_MESH,
    scratch_types=[
        pltpu.VMEM((2, CHK, K), jnp.int32),
        pltpu.VMEM((2, CHK, K), jnp.int32),
        pltpu.VMEM((K, D), jnp.float32),
        pltpu.VMEM((K, D), jnp.float32),
        pltpu.VMEM_SHARED((NP, D), jnp.float32),
        pltpu.SemaphoreType.DMA,
        pltpu.SemaphoreType.DMA,
        pltpu.SemaphoreType.DMA,
        pltpu.SemaphoreType.DMA,
    ],
)


BM = 2048


def _mm2_body(x_ref, w_ref, degp_ref, rmask_ref, o_ref, dv_ref):
  dv0 = lax.rsqrt(degp_ref[0] + 1.0) * rmask_ref[...]
  dv1 = lax.rsqrt(degp_ref[1] + 1.0) * rmask_ref[...]
  dv_ref[0] = dv0
  dv_ref[1] = dv1
  h = jnp.dot(x_ref[...], w_ref[...], preferred_element_type=jnp.float32)
  o_ref[0] = dv0 * h[:, :D]
  o_ref[1] = dv1 * h[:, D:]


_mm2 = pl.pallas_call(
    _mm2_body,
    grid=(NP // BM,),
    in_specs=[
        pl.BlockSpec((BM, D), lambda i: (i, 0)),
        pl.BlockSpec((D, 2 * D), lambda i: (0, 0)),
        pl.BlockSpec((2, BM, 1), lambda i: (0, i, 0)),
        pl.BlockSpec((BM, 1), lambda i: (i, 0)),
    ],
    out_specs=[
        pl.BlockSpec((2, BM, D), lambda i: (0, i, 0)),
        pl.BlockSpec((2, BM, 1), lambda i: (0, i, 0)),
    ],
    out_shape=[
        jax.ShapeDtypeStruct((2, NP, D), jnp.float32),
        jax.ShapeDtypeStruct((2, NP, 1), jnp.float32),
    ],
)


def _mid_body(p_ref, hs_ref, b_ref, dv_ref, w_ref, o_ref):
  t1 = jnp.maximum(
      dv_ref[0] * (p_ref[0] + hs_ref[0]) + b_ref[0], 0.0)
  o_ref[0] = dv_ref[0] * jnp.dot(
      t1, w_ref[0], preferred_element_type=jnp.float32)
  t2 = jnp.maximum(
      dv_ref[1] * (p_ref[1] + hs_ref[1]) + b_ref[1], 0.0)
  o_ref[1] = dv_ref[1] * jnp.dot(
      t2, w_ref[1], preferred_element_type=jnp.float32)


_mid = pl.pallas_call(
    _mid_body,
    grid=(NP // BM,),
    in_specs=[
        pl.BlockSpec((2, BM, D), lambda i: (0, i, 0)),
        pl.BlockSpec((2, BM, D), lambda i: (0, i, 0)),
        pl.BlockSpec((2, 1, D), lambda i: (0, 0, 0)),
        pl.BlockSpec((2, BM, 1), lambda i: (0, i, 0)),
        pl.BlockSpec((2, D, D), lambda i: (0, 0, 0)),
    ],
    out_specs=pl.BlockSpec((2, BM, D), lambda i: (0, i, 0)),
    out_shape=jax.ShapeDtypeStruct((2, NP, D), jnp.float32),
)


BF = 2000


def _fin_body(p_ref, hs_ref, b_ref, dv_ref, o_ref):
  t1 = jnp.maximum(
      dv_ref[0] * (p_ref[0] + hs_ref[0]) + b_ref[0], 0.0)
  t2 = jnp.maximum(
      dv_ref[1] * (p_ref[1] + hs_ref[1]) + b_ref[1], 0.0)
  o_ref[...] = jnp.concatenate([t1, t2], axis=1)


_fin = pl.pallas_call(
    _fin_body,
    grid=(N // BF,),
    in_specs=[
        pl.BlockSpec((2, BF, D), lambda i: (0, i, 0)),
        pl.BlockSpec((2, BF, D), lambda i: (0, i, 0)),
        pl.BlockSpec((2, 1, D), lambda i: (0, 0, 0)),
        pl.BlockSpec((2, BF, 1), lambda i: (0, i, 0)),
    ],
    out_specs=pl.BlockSpec((BF, 2 * D), lambda i: (i, 0)),
    out_shape=jax.ShapeDtypeStruct((N, 2 * D), jnp.float32),
)


def _prep_edges(ei, src_off):
  src = ei[0].astype(jnp.int32)
  dst = ei[1].astype(jnp.int32)
  pad = N + jnp.arange(EP - E, dtype=jnp.int32) % (NP - N)
  src2 = jnp.concatenate([src + src_off, pad + src_off]).reshape(NBLKS, K)
  dst2 = jnp.concatenate([dst, pad]).reshape(NBLKS, K)
  return src2, dst2


def kernel(x, edge_index, edge_index_cross, W1, b1, W2, b2,
           Wc1, bc1, Wc2, bc2):
  f32 = jnp.float32
  xp = jnp.zeros((NP, D), f32).at[:N].set(x.astype(f32))
  s1, t1 = _prep_edges(edge_index, 0)
  s2, t2 = _prep_edges(edge_index_cross, NP)
  scat = jnp.concatenate([s1, s2], axis=0)
  tcat = jnp.concatenate([t1, t2], axis=0)

  degp = _deg(tcat).reshape(2, NP, 1)
  rmask = (jnp.arange(NP) < N).astype(f32).reshape(NP, 1)

  Wcat = jnp.concatenate([W1.astype(f32), Wc1.astype(f32)], axis=1)
  hs1, dv = _mm2(xp, Wcat, degp, rmask)

  p1 = _agg(hs1.reshape(2 * NP, D), scat, tcat).reshape(2, NP, D)

  bcat1 = jnp.stack([b1.reshape(1, D), bc1.reshape(1, D)]).astype(f32)
  bcat2 = jnp.stack([b2.reshape(1, D), bc2.reshape(1, D)]).astype(f32)
  W2cat = jnp.stack([W2.astype(f32), Wc2.astype(f32)])

  hs2 = _mid(p1, hs1, bcat1, dv, W2cat)

  p2 = _agg(hs2.reshape(2 * NP, D), scat, tcat).reshape(2, NP, D)

  return _fin(p2, hs2, bcat2, dv)
